# baseline (speedup 1.0000x reference)
import functools

import jax
import jax.numpy as jnp
from jax import lax
from jax.experimental import pallas as pl
from jax.experimental.pallas import tpu as pltpu
from jax.experimental.pallas import tpu_sc as plsc

VOCAB = 1000
VPAD = 1024
BATCH = 4096
SEQ = 50
SEQ_PAD = 56
NUM_CORES = 2
NUM_SUBCORES = 16
NW = NUM_CORES * NUM_SUBCORES
B_PER_W = BATCH // NW
UPB = SEQ_PAD // 8
NU = B_PER_W * UPB
M = 14
L = 7
NG = NU // M


@functools.partial(
    pl.kernel,
    mesh=plsc.VectorSubcoreMesh(core_axis_name="c", subcore_axis_name="s"),
    out_type=jax.ShapeDtypeStruct((BATCH, SEQ_PAD, VPAD), jnp.float32),
    scratch_types=[
        pltpu.VMEM((B_PER_W * SEQ_PAD,), jnp.int32),
        pltpu.VMEM((M, 8, VPAD), jnp.float32),
        pltpu.SemaphoreType.DMA((M,)),
        pltpu.SemaphoreType.DMA((M,)),
    ],
)
def _gather_rows(x_hbm, table_hbm, out_hbm, idx_v, bufs, sg, sw):
    cid = lax.axis_index("c")
    sid = lax.axis_index("s")
    wid = sid * NUM_CORES + cid
    b0 = wid * B_PER_W

    pltpu.sync_copy(x_hbm.at[pl.ds(b0 * SEQ_PAD, B_PER_W * SEQ_PAD)], idx_v)

    def g(u, r):
        return pltpu.make_async_copy(
            table_hbm.at[idx_v.at[pl.ds(8 * u, 8)]], bufs.at[r], sg.at[r])

    def w(u, r):
        b = u // UPB
        st = u % UPB
        return pltpu.make_async_copy(
            bufs.at[r], out_hbm.at[b0 + b, pl.ds(st * 8, 8), :], sw.at[r])

    for v in range(L):
        g(v, v).start()

    @pl.loop(0, NG)
    def _(grp):
        for r in range(M):
            u = grp * M + r
            rv = (r + L) % M
            v = u + L
            g(u, r).wait()
            w(u, r).start()

            @pl.when(v < NU)
            def _():
                @pl.when(v >= M)
                def _():
                    w(v - M, rv).wait()
                g(v, rv).start()

    for r in range(M):
        w(NU - M + r, r).wait()


def kernel(x, bigram):
    xp = jnp.pad(x.astype(jnp.int32), ((0, 0), (0, SEQ_PAD - SEQ)))
    tp = jnp.pad(bigram, ((0, 0), (0, VPAD - VOCAB)))
    out = _gather_rows(xp.reshape(-1), tp)
    return out[:, :SEQ, :VOCAB]


# speedup vs baseline: 1.6954x; 1.6954x over previous
import functools

import jax
import jax.numpy as jnp
from jax import lax
from jax.experimental import pallas as pl
from jax.experimental.pallas import tpu as pltpu
from jax.experimental.pallas import tpu_sc as plsc

VOCAB = 1000
VPAD = 1024
BATCH = 4096
SEQ = 50
SEQ_PAD = 56
NUM_CORES = 2
NUM_SUBCORES = 16
NW = NUM_CORES * NUM_SUBCORES
B_PER_W = BATCH // NW
NA = 24
NB = 32
NB_VALID = SEQ - NA


@functools.partial(
    pl.kernel,
    mesh=plsc.VectorSubcoreMesh(core_axis_name="c", subcore_axis_name="s"),
    compiler_params=pltpu.CompilerParams(use_tc_tiling_on_sc=False),
    out_type=jax.ShapeDtypeStruct((BATCH, SEQ, VPAD), jnp.float32),
    scratch_types=[
        pltpu.VMEM_SHARED((VOCAB, VPAD), jnp.float32),
        pltpu.VMEM((B_PER_W * SEQ_PAD,), jnp.int32),
        pltpu.VMEM((NA, VPAD), jnp.float32),
        pltpu.VMEM((NB, VPAD), jnp.float32),
        pltpu.SemaphoreType.DMA,
        pltpu.SemaphoreType.DMA,
        pltpu.SemaphoreType.DMA,
        pltpu.SemaphoreType.DMA,
    ],
)
def _gather_rows(x_hbm, table_hbm, out_hbm, shared, idx_v, buf_a, buf_b,
                 sga, sgb, swa, swb):
    cid = lax.axis_index("c")
    sid = lax.axis_index("s")
    wid = sid * NUM_CORES + cid
    b0 = wid * B_PER_W

    @pl.when(sid < 8)
    def _():
        pltpu.sync_copy(table_hbm.at[pl.ds(sid * 125, 125)],
                        shared.at[pl.ds(sid * 125, 125)])
    pltpu.sync_copy(x_hbm.at[pl.ds(b0 * SEQ_PAD, B_PER_W * SEQ_PAD)], idx_v)
    plsc.subcore_barrier()

    def ga(b):
        return pltpu.make_async_copy(
            shared.at[idx_v.at[pl.ds(b * SEQ_PAD, NA)]], buf_a, sga)

    def gb(b):
        return pltpu.make_async_copy(
            shared.at[idx_v.at[pl.ds(b * SEQ_PAD + NA, NB)]], buf_b, sgb)

    def wa(b):
        return pltpu.make_async_copy(
            buf_a, out_hbm.at[b0 + b, pl.ds(0, NA), :], swa)

    def wb(b):
        return pltpu.make_async_copy(
            buf_b.at[pl.ds(0, NB_VALID)],
            out_hbm.at[b0 + b, pl.ds(NA, NB_VALID), :], swb)

    ga(0).start()
    gb(0).start()

    def body(b, carry):
        ga(b).wait()
        wa(b).start()
        gb(b).wait()
        wb(b).start()

        @pl.when(b < B_PER_W - 1)
        def _():
            wa(b).wait()
            ga(b + 1).start()
            wb(b).wait()
            gb(b + 1).start()
        return carry

    lax.fori_loop(0, B_PER_W, body, 0)
    wa(B_PER_W - 1).wait()
    wb(B_PER_W - 1).wait()


def kernel(x, bigram):
    xp = jnp.pad(x.astype(jnp.int32), ((0, 0), (0, SEQ_PAD - SEQ)))
    tp = jnp.pad(bigram, ((0, 0), (0, VPAD - VOCAB)))
    out = _gather_rows(xp.reshape(-1), tp)
    return out[:, :, :VOCAB]
